# baseline (speedup 1.0000x reference)
import functools

import jax
import jax.numpy as jnp
from jax import lax
from jax.experimental import pallas as pl
from jax.experimental.pallas import tpu as pltpu
from jax.experimental.pallas import tpu_sc as plsc

_VOCAB = 100000
_E = 64
_H = 256
_O = 5
_B = 16384
_L = 200
_LH = 100
_NC = 2
_NS = 16
_NW = _NC * _NS
_VP = 102400
_COLS = -(-_LH // _NW)
_KB = 12800


def _hist_body(start, xT, counts, idx_v, cnt_v):
    wid = lax.axis_index("s") * _NC + lax.axis_index("c")

    @plsc.parallel_loop(0, _VP, 16, unroll=8)
    def _zero(i):
        cnt_v[pl.ds(i, 16)] = jnp.zeros((16,), jnp.float32)

    ones = jnp.ones((16,), jnp.float32)
    zeros = jnp.zeros((16,), jnp.float32)

    def _col(j, c):
        r = wid + j * _NW
        l = start + r

        @pl.when(r < _LH)
        def _():
            pltpu.sync_copy(xT.at[l], idx_v)

            @plsc.parallel_loop(0, _B, 16, unroll=8)
            def _scat(i):
                v = idx_v[pl.ds(i, 16)]
                plsc.addupdate_scatter(cnt_v, [v], ones)

            pltpu.sync_copy(cnt_v, counts.at[r])

            @plsc.parallel_loop(0, _B, 16, unroll=8)
            def _unscat(i):
                v = idx_v[pl.ds(i, 16)]
                plsc.store_scatter(cnt_v, [v], zeros)
        return c

    lax.fori_loop(0, _COLS, _col, 0)


def _make_hist(start):
    return functools.partial(
        pl.kernel,
        out_type=jax.ShapeDtypeStruct((_LH, _VP), jnp.float32),
        mesh=plsc.VectorSubcoreMesh(core_axis_name="c", subcore_axis_name="s"),
        scratch_types=[
            pltpu.VMEM((_B,), jnp.int32),
            pltpu.VMEM((_VP,), jnp.float32),
        ],
        compiler_params=pltpu.CompilerParams(needs_layout_passes=False),
    )(functools.partial(_hist_body, start))


_hist0 = _make_hist(0)
_hist1 = _make_hist(_LH)


def _mm_body(counts_ref, emb_ref, s_ref):
    k = pl.program_id(0)

    @pl.when(k == 0)
    def _():
        s_ref[...] = jnp.zeros_like(s_ref)

    s_ref[...] += jnp.dot(counts_ref[...], emb_ref[...],
                          preferred_element_type=jnp.float32)


_mm = pl.pallas_call(
    _mm_body,
    grid=(_VP // _KB,),
    in_specs=[
        pl.BlockSpec((_LH, _KB), lambda k: (0, k)),
        pl.BlockSpec((_KB, _E), lambda k: (k, 0)),
    ],
    out_specs=pl.BlockSpec((_LH, _E), lambda k: (0, 0)),
    out_shape=jax.ShapeDtypeStruct((_LH, _E), jnp.float32),
)


def _tail_body(counts_ref, emb_ref, s1_ref, w1t_ref, b1_ref, w2t_ref, b2_ref,
               out_ref, acc_ref):
    k = pl.program_id(0)

    @pl.when(k == 0)
    def _():
        acc_ref[...] = jnp.zeros_like(acc_ref)

    acc_ref[...] += jnp.dot(counts_ref[...], emb_ref[...],
                            preferred_element_type=jnp.float32)

    @pl.when(k == pl.num_programs(0) - 1)
    def _():
        s = jnp.concatenate([s1_ref[...], acc_ref[...]], axis=0) * (1.0 / _B)
        h1 = jnp.tanh(s @ w1t_ref[...] + b1_ref[...])
        h2 = h1 @ w2t_ref[...] + b2_ref[...]
        m = jnp.max(h2, axis=0, keepdims=True)
        lse = jnp.log(jnp.sum(jnp.exp(h2 - m), axis=0, keepdims=True)) + m
        out_ref[...] = h2 - lse


_tail = pl.pallas_call(
    _tail_body,
    grid=(_VP // _KB,),
    in_specs=[
        pl.BlockSpec((_LH, _KB), lambda k: (0, k)),
        pl.BlockSpec((_KB, _E), lambda k: (k, 0)),
        pl.BlockSpec((_LH, _E), lambda k: (0, 0)),
        pl.BlockSpec((_E, _H), lambda k: (0, 0)),
        pl.BlockSpec((1, _H), lambda k: (0, 0)),
        pl.BlockSpec((_H, _O), lambda k: (0, 0)),
        pl.BlockSpec((1, _O), lambda k: (0, 0)),
    ],
    out_specs=pl.BlockSpec((_L, _O), lambda k: (0, 0)),
    out_shape=jax.ShapeDtypeStruct((_L, _O), jnp.float32),
    scratch_shapes=[pltpu.VMEM((_LH, _E), jnp.float32)],
)


def kernel(x, emb, W1, b1, W2, b2):
    xT = x.T.astype(jnp.int32)
    emb_p = jnp.zeros((_VP, _E), jnp.float32).at[:_VOCAB].set(emb)
    counts1 = _hist0(xT)
    s1 = _mm(counts1, emb_p)
    counts2 = _hist1(xT)
    return _tail(counts2, emb_p, s1, W1.T, b1.reshape(1, _H),
                 W2.T, b2.reshape(1, _O))


# speedup vs baseline: 1.3611x; 1.3611x over previous
import functools

import jax
import jax.numpy as jnp
from jax import lax
from jax.experimental import pallas as pl
from jax.experimental.pallas import tpu as pltpu
from jax.experimental.pallas import tpu_sc as plsc

_VOCAB = 100000
_E = 64
_H = 256
_O = 5
_B = 16384
_L = 200
_NC = 2
_NS = 16
_NW = _NC * _NS
_VP = 102400
_HALF = _VP // 2
_COLS = -(-_L // _NW)
_KW = 6400


def _hist_body(xT, counts, idx_v, cnt_v):
    wid = lax.axis_index("s") * _NC + lax.axis_index("c")

    @plsc.parallel_loop(0, _HALF, 16, unroll=8)
    def _zero(i):
        cnt_v[pl.ds(i, 16)] = jnp.zeros((16,), jnp.int32)

    zeros = jnp.zeros((16,), jnp.int32)

    def _col(j, c):
        l = wid + j * _NW

        @pl.when(l < _L)
        def _():
            pltpu.sync_copy(xT.at[l], idx_v)

            @plsc.parallel_loop(0, _B, 16, unroll=8)
            def _scat(i):
                v = idx_v[pl.ds(i, 16)]
                hi = (v >= _HALF).astype(jnp.int32)
                a = v - hi * _HALF
                w = jnp.int32(1) << (hi << 4)
                plsc.addupdate_scatter(cnt_v, [a], w)

            pltpu.sync_copy(cnt_v, counts.at[l])

            @plsc.parallel_loop(0, _B, 16, unroll=8)
            def _unscat(i):
                v = idx_v[pl.ds(i, 16)]
                hi = (v >= _HALF).astype(jnp.int32)
                a = v - hi * _HALF
                plsc.store_scatter(cnt_v, [a], zeros)
        return c

    lax.fori_loop(0, _COLS, _col, 0)


_hist = functools.partial(
    pl.kernel,
    out_type=jax.ShapeDtypeStruct((_L, _HALF), jnp.int32),
    mesh=plsc.VectorSubcoreMesh(core_axis_name="c", subcore_axis_name="s"),
    scratch_types=[
        pltpu.VMEM((_B,), jnp.int32),
        pltpu.VMEM((_HALF,), jnp.int32),
    ],
    compiler_params=pltpu.CompilerParams(needs_layout_passes=False),
)(_hist_body)


def _mlp_body(counts_ref, emb_lo_ref, emb_hi_ref, w1t_ref, b1_ref,
              w2t_ref, b2_ref, out_ref, acc_ref):
    k = pl.program_id(0)

    @pl.when(k == 0)
    def _():
        acc_ref[...] = jnp.zeros_like(acc_ref)

    blk = counts_ref[...]
    lo = (blk & 0xFFFF).astype(jnp.float32)
    hi = (blk >> 16).astype(jnp.float32)
    acc_ref[...] += (jnp.dot(lo, emb_lo_ref[...],
                             preferred_element_type=jnp.float32) +
                     jnp.dot(hi, emb_hi_ref[...],
                             preferred_element_type=jnp.float32))

    @pl.when(k == pl.num_programs(0) - 1)
    def _():
        s = acc_ref[...] * (1.0 / _B)
        h1 = jnp.tanh(s @ w1t_ref[...] + b1_ref[...])
        h2 = h1 @ w2t_ref[...] + b2_ref[...]
        m = jnp.max(h2, axis=0, keepdims=True)
        lse = jnp.log(jnp.sum(jnp.exp(h2 - m), axis=0, keepdims=True)) + m
        out_ref[...] = h2 - lse


_mlp = pl.pallas_call(
    _mlp_body,
    grid=(_HALF // _KW,),
    in_specs=[
        pl.BlockSpec((_L, _KW), lambda k: (0, k)),
        pl.BlockSpec((_KW, _E), lambda k: (k, 0)),
        pl.BlockSpec((_KW, _E), lambda k: (k + _HALF // _KW, 0)),
        pl.BlockSpec((_E, _H), lambda k: (0, 0)),
        pl.BlockSpec((1, _H), lambda k: (0, 0)),
        pl.BlockSpec((_H, _O), lambda k: (0, 0)),
        pl.BlockSpec((1, _O), lambda k: (0, 0)),
    ],
    out_specs=pl.BlockSpec((_L, _O), lambda k: (0, 0)),
    out_shape=jax.ShapeDtypeStruct((_L, _O), jnp.float32),
    scratch_shapes=[pltpu.VMEM((_L, _E), jnp.float32)],
)


def kernel(x, emb, W1, b1, W2, b2):
    xT = x.T.astype(jnp.int32)
    counts = _hist(xT)
    emb_p = jnp.zeros((_VP, _E), jnp.float32).at[:_VOCAB].set(emb)
    return _mlp(counts, emb_p, emb_p, W1.T, b1.reshape(1, _H),
                W2.T, b2.reshape(1, _O))


# speedup vs baseline: 1.3823x; 1.0156x over previous
import functools

import jax
import jax.numpy as jnp
from jax import lax
from jax.experimental import pallas as pl
from jax.experimental.pallas import tpu as pltpu
from jax.experimental.pallas import tpu_sc as plsc

_VOCAB = 100000
_E = 64
_H = 256
_O = 5
_B = 16384
_L = 200
_NC = 2
_NS = 16
_NW = _NC * _NS
_VP = 102400
_HALF = _VP // 2
_COLS = -(-_L // _NW)
_KW = 6400


def _hist_body(xT, counts, idx_v, cnt_a, cnt_b, sem_a, sem_b):
    wid = lax.axis_index("s") * _NC + lax.axis_index("c")
    pend = [None, None]

    for j in range(_COLS):
        slot = j % 2
        buf, sem = (cnt_a, sem_a) if slot == 0 else (cnt_b, sem_b)
        l = wid + j * _NW
        cond = l < _L

        @pl.when(cond)
        def _(buf=buf, sem=sem, l=l, slot=slot):
            if pend[slot] is not None:
                pend[slot][0].wait()

            @plsc.parallel_loop(0, _HALF, 16, unroll=8)
            def _zero(i):
                buf[pl.ds(i, 16)] = jnp.zeros((16,), jnp.int32)

            pltpu.sync_copy(xT.at[l], idx_v)

            @plsc.parallel_loop(0, _B, 16, unroll=8)
            def _scat(i):
                v = idx_v[pl.ds(i, 16)]
                hi = (v >= _HALF).astype(jnp.int32)
                a = v - hi * _HALF
                w = jnp.int32(1) << (hi << 4)
                plsc.addupdate_scatter(buf, [a], w)

            pend[slot] = (pltpu.async_copy(buf, counts.at[l], sem), cond)

    for slot in (0, 1):
        if pend[slot] is not None:
            desc, cond = pend[slot]

            @pl.when(cond)
            def _(desc=desc):
                desc.wait()


_hist = functools.partial(
    pl.kernel,
    out_type=jax.ShapeDtypeStruct((_L, _HALF), jnp.int32),
    mesh=plsc.VectorSubcoreMesh(core_axis_name="c", subcore_axis_name="s"),
    scratch_types=[
        pltpu.VMEM((_B,), jnp.int32),
        pltpu.VMEM((_HALF,), jnp.int32),
        pltpu.VMEM((_HALF,), jnp.int32),
        pltpu.SemaphoreType.DMA,
        pltpu.SemaphoreType.DMA,
    ],
    compiler_params=pltpu.CompilerParams(needs_layout_passes=False),
)(_hist_body)


def _mlp_body(counts_ref, emb_lo_ref, emb_hi_ref, w1t_ref, b1_ref,
              w2t_ref, b2_ref, out_ref, acc_ref):
    k = pl.program_id(0)

    @pl.when(k == 0)
    def _():
        acc_ref[...] = jnp.zeros_like(acc_ref)

    blk = counts_ref[...]
    lo = (blk & 0xFFFF).astype(jnp.float32)
    hi = (blk >> 16).astype(jnp.float32)
    acc_ref[...] += (jnp.dot(lo, emb_lo_ref[...],
                             preferred_element_type=jnp.float32) +
                     jnp.dot(hi, emb_hi_ref[...],
                             preferred_element_type=jnp.float32))

    @pl.when(k == pl.num_programs(0) - 1)
    def _():
        s = acc_ref[...] * (1.0 / _B)
        h1 = jnp.tanh(s @ w1t_ref[...] + b1_ref[...])
        h2 = h1 @ w2t_ref[...] + b2_ref[...]
        m = jnp.max(h2, axis=0, keepdims=True)
        lse = jnp.log(jnp.sum(jnp.exp(h2 - m), axis=0, keepdims=True)) + m
        out_ref[...] = h2 - lse


_mlp = pl.pallas_call(
    _mlp_body,
    grid=(_HALF // _KW,),
    in_specs=[
        pl.BlockSpec((_L, _KW), lambda k: (0, k)),
        pl.BlockSpec((_KW, _E), lambda k: (k, 0)),
        pl.BlockSpec((_KW, _E), lambda k: (k + _HALF // _KW, 0)),
        pl.BlockSpec((_E, _H), lambda k: (0, 0)),
        pl.BlockSpec((1, _H), lambda k: (0, 0)),
        pl.BlockSpec((_H, _O), lambda k: (0, 0)),
        pl.BlockSpec((1, _O), lambda k: (0, 0)),
    ],
    out_specs=pl.BlockSpec((_L, _O), lambda k: (0, 0)),
    out_shape=jax.ShapeDtypeStruct((_L, _O), jnp.float32),
    scratch_shapes=[pltpu.VMEM((_L, _E), jnp.float32)],
)


def kernel(x, emb, W1, b1, W2, b2):
    xT = x.T.astype(jnp.int32)
    counts = _hist(xT)
    emb_p = jnp.zeros((_VP, _E), jnp.float32).at[:_VOCAB].set(emb)
    return _mlp(counts, emb_p, emb_p, W1.T, b1.reshape(1, _H),
                W2.T, b2.reshape(1, _O))


# speedup vs baseline: 1.4180x; 1.0258x over previous
import functools

import jax
import jax.numpy as jnp
from jax import lax
from jax.experimental import pallas as pl
from jax.experimental.pallas import tpu as pltpu
from jax.experimental.pallas import tpu_sc as plsc

_VOCAB = 100000
_E = 64
_H = 256
_O = 5
_B = 16384
_L = 200
_NC = 2
_NS = 16
_NW = _NC * _NS
_VP = 102400
_HALF = _VP // 2
_COLS = -(-_L // _NW)
_KW = 6400


def _hist_body(xT, counts, idx_v, cnt_a, cnt_b, sem_a, sem_b):
    wid = lax.axis_index("s") * _NC + lax.axis_index("c")
    issued = [[], []]

    for j in range(_COLS):
        slot = j % 2
        buf, sem = (cnt_a, sem_a) if slot == 0 else (cnt_b, sem_b)
        l = wid + j * _NW
        cond = l < _L

        @pl.when(cond)
        def _(buf=buf, sem=sem, l=l, slot=slot):
            if issued[slot]:
                issued[slot][-1][0].wait()
                issued[slot][-1][2] = cond

            @plsc.parallel_loop(0, _HALF, 16, unroll=8)
            def _zero(i):
                buf[pl.ds(i, 16)] = jnp.zeros((16,), jnp.int32)

            pltpu.sync_copy(xT.at[l], idx_v)

            @plsc.parallel_loop(0, _B, 16, unroll=8)
            def _scat(i):
                v = idx_v[pl.ds(i, 16)]
                hi = (v >= _HALF).astype(jnp.int32)
                a = v - hi * _HALF
                w = jnp.int32(1) << (hi << 4)
                plsc.addupdate_scatter(buf, [a], w)

            issued[slot].append([pltpu.async_copy(buf, counts.at[l], sem),
                                 cond, None])

    for slot in (0, 1):
        for desc, icond, dcond in issued[slot]:
            rem = icond if dcond is None else icond & jnp.logical_not(dcond)

            @pl.when(rem)
            def _(desc=desc):
                desc.wait()


_hist = functools.partial(
    pl.kernel,
    out_type=jax.ShapeDtypeStruct((_L, _HALF), jnp.int32),
    mesh=plsc.VectorSubcoreMesh(core_axis_name="c", subcore_axis_name="s"),
    scratch_types=[
        pltpu.VMEM((_B,), jnp.int32),
        pltpu.VMEM((_HALF,), jnp.int32),
        pltpu.VMEM((_HALF,), jnp.int32),
        pltpu.SemaphoreType.DMA,
        pltpu.SemaphoreType.DMA,
    ],
    compiler_params=pltpu.CompilerParams(needs_layout_passes=False),
)(_hist_body)


def _mlp_body(counts_ref, emb_lo_ref, emb_hi_ref, w1t_ref, b1_ref,
              w2t_ref, b2_ref, out_ref, acc_ref):
    k = pl.program_id(0)

    @pl.when(k == 0)
    def _():
        acc_ref[...] = jnp.zeros_like(acc_ref)

    blk = counts_ref[...]
    lo = (blk & 0xFFFF).astype(jnp.float32)
    hi = (blk >> 16).astype(jnp.float32)
    acc_ref[...] += (jnp.dot(lo, emb_lo_ref[...],
                             preferred_element_type=jnp.float32) +
                     jnp.dot(hi, emb_hi_ref[...],
                             preferred_element_type=jnp.float32))

    @pl.when(k == pl.num_programs(0) - 1)
    def _():
        s = acc_ref[...] * (1.0 / _B)
        h1 = jnp.tanh(s @ w1t_ref[...] + b1_ref[...])
        h2 = h1 @ w2t_ref[...] + b2_ref[...]
        m = jnp.max(h2, axis=0, keepdims=True)
        lse = jnp.log(jnp.sum(jnp.exp(h2 - m), axis=0, keepdims=True)) + m
        out_ref[...] = h2 - lse


_mlp = pl.pallas_call(
    _mlp_body,
    grid=(_HALF // _KW,),
    in_specs=[
        pl.BlockSpec((_L, _KW), lambda k: (0, k)),
        pl.BlockSpec((_KW, _E), lambda k: (k, 0)),
        pl.BlockSpec((_KW, _E), lambda k: (k + _HALF // _KW, 0)),
        pl.BlockSpec((_E, _H), lambda k: (0, 0)),
        pl.BlockSpec((1, _H), lambda k: (0, 0)),
        pl.BlockSpec((_H, _O), lambda k: (0, 0)),
        pl.BlockSpec((1, _O), lambda k: (0, 0)),
    ],
    out_specs=pl.BlockSpec((_L, _O), lambda k: (0, 0)),
    out_shape=jax.ShapeDtypeStruct((_L, _O), jnp.float32),
    scratch_shapes=[pltpu.VMEM((_L, _E), jnp.float32)],
)


def kernel(x, emb, W1, b1, W2, b2):
    xT = x.T.astype(jnp.int32)
    counts = _hist(xT)
    emb_p = jnp.zeros((_VP, _E), jnp.float32).at[:_VOCAB].set(emb)
    return _mlp(counts, emb_p, emb_p, W1.T, b1.reshape(1, _H),
                W2.T, b2.reshape(1, _O))


# speedup vs baseline: 1.5381x; 1.0847x over previous
import functools

import jax
import jax.numpy as jnp
from jax import lax
from jax.experimental import pallas as pl
from jax.experimental.pallas import tpu as pltpu
from jax.experimental.pallas import tpu_sc as plsc

_VOCAB = 100000
_E = 64
_H = 256
_O = 5
_B = 16384
_L = 200
_LH = 100
_NC = 2
_NS = 16
_NW = _NC * _NS
_VP = 102400
_HALF = _VP // 2
_COLS = -(-_LH // _NW)
_KW = 6400
_KS = _HALF // _KW
_TAIL = _VOCAB - (_HALF + (_KS - 1) * _KW)


def _hist_body(start, xT, counts, idx_v, cnt_a, cnt_b, sem_a, sem_b):
    wid = lax.axis_index("s") * _NC + lax.axis_index("c")
    issued = [[], []]

    for j in range(_COLS):
        slot = j % 2
        buf, sem = (cnt_a, sem_a) if slot == 0 else (cnt_b, sem_b)
        r = wid + j * _NW
        cond = r < _LH

        @pl.when(cond)
        def _(buf=buf, sem=sem, r=r):
            if issued[slot]:
                issued[slot][-1][0].wait()
                issued[slot][-1][2] = cond

            @plsc.parallel_loop(0, _HALF, 16, unroll=8)
            def _zero(i):
                buf[pl.ds(i, 16)] = jnp.zeros((16,), jnp.int32)

            pltpu.sync_copy(xT.at[start + r], idx_v)

            @plsc.parallel_loop(0, _B, 16, unroll=8)
            def _scat(i):
                v = idx_v[pl.ds(i, 16)]
                hi = (v >= _HALF).astype(jnp.int32)
                a = v - hi * _HALF
                w = jnp.int32(1) << (hi << 4)
                plsc.addupdate_scatter(buf, [a], w)

            issued[slot].append([pltpu.async_copy(buf, counts.at[r], sem),
                                 cond, None])

    for slot in (0, 1):
        for desc, icond, dcond in issued[slot]:
            rem = icond if dcond is None else icond & jnp.logical_not(dcond)

            @pl.when(rem)
            def _(desc=desc):
                desc.wait()


def _make_hist(start):
    return functools.partial(
        pl.kernel,
        out_type=jax.ShapeDtypeStruct((_LH, _HALF), jnp.int32),
        mesh=plsc.VectorSubcoreMesh(core_axis_name="c", subcore_axis_name="s"),
        scratch_types=[
            pltpu.VMEM((_B,), jnp.int32),
            pltpu.VMEM((_HALF,), jnp.int32),
            pltpu.VMEM((_HALF,), jnp.int32),
            pltpu.SemaphoreType.DMA,
            pltpu.SemaphoreType.DMA,
        ],
        compiler_params=pltpu.CompilerParams(needs_layout_passes=False),
    )(functools.partial(_hist_body, start))


_hist0 = _make_hist(0)
_hist1 = _make_hist(_LH)


def _accum_step(counts_ref, emb_lo_ref, emb_hi_ref, emb_tail_ref, acc_ref):
    k = pl.program_id(0)
    blk = counts_ref[...]
    lo = (blk & 0xFFFF).astype(jnp.float32)
    hi = (blk >> 16).astype(jnp.float32)
    acc_ref[...] += jnp.dot(lo, emb_lo_ref[...],
                            preferred_element_type=jnp.float32)

    @pl.when(k < _KS - 1)
    def _():
        acc_ref[...] += jnp.dot(hi, emb_hi_ref[...],
                                preferred_element_type=jnp.float32)

    @pl.when(k == _KS - 1)
    def _():
        acc_ref[...] += jnp.dot(hi[:, :_TAIL], emb_tail_ref[...],
                                preferred_element_type=jnp.float32)


def _mm_body(counts_ref, emb_lo_ref, emb_hi_ref, emb_tail_ref, s_ref):
    @pl.when(pl.program_id(0) == 0)
    def _():
        s_ref[...] = jnp.zeros_like(s_ref)

    _accum_step(counts_ref, emb_lo_ref, emb_hi_ref, emb_tail_ref, s_ref)


def _tail_body(counts_ref, emb_lo_ref, emb_hi_ref, emb_tail_ref, s1_ref,
               w1t_ref, b1_ref, w2t_ref, b2_ref, out_ref, acc_ref):
    k = pl.program_id(0)

    @pl.when(k == 0)
    def _():
        acc_ref[...] = jnp.zeros_like(acc_ref)

    _accum_step(counts_ref, emb_lo_ref, emb_hi_ref, emb_tail_ref, acc_ref)

    @pl.when(k == pl.num_programs(0) - 1)
    def _():
        s = jnp.concatenate([s1_ref[...], acc_ref[...]], axis=0) * (1.0 / _B)
        h1 = jnp.tanh(s @ w1t_ref[...] + b1_ref[...])
        h2 = h1 @ w2t_ref[...] + b2_ref[...]
        m = jnp.max(h2, axis=0, keepdims=True)
        lse = jnp.log(jnp.sum(jnp.exp(h2 - m), axis=0, keepdims=True)) + m
        out_ref[...] = h2 - lse


_emb_specs = [
    pl.BlockSpec((_KW, _E), lambda k: (k, 0)),
    pl.BlockSpec((_KW, _E), lambda k: (jnp.minimum(k + _KS, 2 * _KS - 2), 0)),
    pl.BlockSpec((_TAIL, _E), lambda k: (0, 0)),
]

_mm = pl.pallas_call(
    _mm_body,
    grid=(_KS,),
    in_specs=[pl.BlockSpec((_LH, _KW), lambda k: (0, k))] + _emb_specs,
    out_specs=pl.BlockSpec((_LH, _E), lambda k: (0, 0)),
    out_shape=jax.ShapeDtypeStruct((_LH, _E), jnp.float32),
)

_tail = pl.pallas_call(
    _tail_body,
    grid=(_KS,),
    in_specs=[pl.BlockSpec((_LH, _KW), lambda k: (0, k))] + _emb_specs + [
        pl.BlockSpec((_LH, _E), lambda k: (0, 0)),
        pl.BlockSpec((_E, _H), lambda k: (0, 0)),
        pl.BlockSpec((1, _H), lambda k: (0, 0)),
        pl.BlockSpec((_H, _O), lambda k: (0, 0)),
        pl.BlockSpec((1, _O), lambda k: (0, 0)),
    ],
    out_specs=pl.BlockSpec((_L, _O), lambda k: (0, 0)),
    out_shape=jax.ShapeDtypeStruct((_L, _O), jnp.float32),
    scratch_shapes=[pltpu.VMEM((_LH, _E), jnp.float32)],
)


def kernel(x, emb, W1, b1, W2, b2):
    xT = x.T.astype(jnp.int32)
    emb_tail = emb[_HALF + (_KS - 1) * _KW:_VOCAB]
    counts1 = _hist0(xT)
    s1 = _mm(counts1, emb, emb, emb_tail)
    counts2 = _hist1(xT)
    return _tail(counts2, emb, emb, emb_tail, s1, W1.T, b1.reshape(1, _H),
                 W2.T, b2.reshape(1, _O))


# speedup vs baseline: 1.5494x; 1.0073x over previous
import functools

import jax
import jax.numpy as jnp
from jax import lax
from jax.experimental import pallas as pl
from jax.experimental.pallas import tpu as pltpu
from jax.experimental.pallas import tpu_sc as plsc

_VOCAB = 100000
_E = 64
_H = 256
_O = 5
_B = 16384
_L = 200
_LH = 100
_NC = 2
_NS = 16
_NW = _NC * _NS
_VP = 102400
_HALF = _VP // 2
_COLS = -(-_LH // _NW)
_KW = 6400
_KS = _HALF // _KW
_TAIL = _VOCAB - (_HALF + (_KS - 1) * _KW)


def _hist_body(start, xT, counts, idx_v, cnt_a, cnt_b, sem_a, sem_b):
    wid = lax.axis_index("s") * _NC + lax.axis_index("c")
    issued = [[], []]

    for j in range(_COLS):
        slot = j % 2
        buf, sem = (cnt_a, sem_a) if slot == 0 else (cnt_b, sem_b)
        r = wid + j * _NW
        cond = r < _LH

        @pl.when(cond)
        def _(buf=buf, sem=sem, r=r):
            if issued[slot]:
                issued[slot][-1][0].wait()
                issued[slot][-1][2] = cond

            @plsc.parallel_loop(0, _HALF, 16, unroll=8)
            def _zero(i):
                buf[pl.ds(i, 16)] = jnp.zeros((16,), jnp.int32)

            pltpu.sync_copy(xT.at[start + r], idx_v)

            @plsc.parallel_loop(0, _B, 16, unroll=8)
            def _scat(i):
                v = idx_v[pl.ds(i, 16)]
                hi = (v >= _HALF).astype(jnp.int32)
                a = v - hi * _HALF
                w = jnp.int32(1) << (hi << 4)
                plsc.addupdate_scatter(buf, [a], w)

            issued[slot].append([pltpu.async_copy(buf, counts.at[r], sem),
                                 cond, None])

    for slot in (0, 1):
        for desc, icond, dcond in issued[slot]:
            rem = icond if dcond is None else icond & jnp.logical_not(dcond)

            @pl.when(rem)
            def _(desc=desc):
                desc.wait()


def _make_hist(start):
    return functools.partial(
        pl.kernel,
        out_type=jax.ShapeDtypeStruct((_LH, _HALF), jnp.int32),
        mesh=plsc.VectorSubcoreMesh(core_axis_name="c", subcore_axis_name="s"),
        scratch_types=[
            pltpu.VMEM((_B,), jnp.int32),
            pltpu.VMEM((_HALF,), jnp.int32),
            pltpu.VMEM((_HALF,), jnp.int32),
            pltpu.SemaphoreType.DMA,
            pltpu.SemaphoreType.DMA,
        ],
        compiler_params=pltpu.CompilerParams(needs_layout_passes=False),
    )(functools.partial(_hist_body, start))


_hist0 = _make_hist(0)
_hist1 = _make_hist(_LH)


def _accum_step(counts_ref, emb_lo_ref, emb_hi_ref, emb_tail_ref, acc_ref):
    k = pl.program_id(0)
    blk = counts_ref[...]
    lo = (blk & 0xFFFF).astype(jnp.float32)
    hi = (blk >> 16).astype(jnp.float32)
    acc_ref[...] += jnp.dot(lo, emb_lo_ref[...],
                            preferred_element_type=jnp.float32)

    @pl.when(k < _KS - 1)
    def _():
        acc_ref[...] += jnp.dot(hi, emb_hi_ref[...],
                                preferred_element_type=jnp.float32)

    @pl.when(k == _KS - 1)
    def _():
        acc_ref[...] += jnp.dot(hi[:, :_TAIL], emb_tail_ref[...],
                                preferred_element_type=jnp.float32)


def _mm_body(counts_ref, emb_lo_ref, emb_hi_ref, emb_tail_ref, s_ref):
    @pl.when(pl.program_id(0) == 0)
    def _():
        s_ref[...] = jnp.zeros_like(s_ref)

    _accum_step(counts_ref, emb_lo_ref, emb_hi_ref, emb_tail_ref, s_ref)


def _tail_body(counts_ref, emb_lo_ref, emb_hi_ref, emb_tail_ref, s1_ref,
               w1t_ref, b1_ref, w2t_ref, b2_ref, out_ref, acc_ref):
    k = pl.program_id(0)

    @pl.when(k == 0)
    def _():
        acc_ref[...] = jnp.zeros_like(acc_ref)

    _accum_step(counts_ref, emb_lo_ref, emb_hi_ref, emb_tail_ref, acc_ref)

    @pl.when(k == pl.num_programs(0) - 1)
    def _():
        s = jnp.concatenate([s1_ref[...], acc_ref[...]], axis=0) * (1.0 / _B)
        h1 = jnp.tanh(s @ w1t_ref[...] + b1_ref[...])
        h2 = h1 @ w2t_ref[...] + b2_ref[...]
        m = jnp.max(h2, axis=0, keepdims=True)
        lse = jnp.log(jnp.sum(jnp.exp(h2 - m), axis=0, keepdims=True)) + m
        out_ref[...] = h2 - lse


_emb_specs = [
    pl.BlockSpec((_KW, _E), lambda k: (k, 0)),
    pl.BlockSpec((_KW, _E), lambda k: (jnp.minimum(k + _KS, 2 * _KS - 2), 0)),
    pl.BlockSpec((_TAIL, _E), lambda k: (_VOCAB // _TAIL - 1, 0)),
]

_mm = pl.pallas_call(
    _mm_body,
    grid=(_KS,),
    in_specs=[pl.BlockSpec((_LH, _KW), lambda k: (0, k))] + _emb_specs,
    out_specs=pl.BlockSpec((_LH, _E), lambda k: (0, 0)),
    out_shape=jax.ShapeDtypeStruct((_LH, _E), jnp.float32),
)

_tail = pl.pallas_call(
    _tail_body,
    grid=(_KS,),
    in_specs=[pl.BlockSpec((_LH, _KW), lambda k: (0, k))] + _emb_specs + [
        pl.BlockSpec((_LH, _E), lambda k: (0, 0)),
        pl.BlockSpec((_E, _H), lambda k: (0, 0)),
        pl.BlockSpec((1, _H), lambda k: (0, 0)),
        pl.BlockSpec((_H, _O), lambda k: (0, 0)),
        pl.BlockSpec((1, _O), lambda k: (0, 0)),
    ],
    out_specs=pl.BlockSpec((_L, _O), lambda k: (0, 0)),
    out_shape=jax.ShapeDtypeStruct((_L, _O), jnp.float32),
    scratch_shapes=[pltpu.VMEM((_LH, _E), jnp.float32)],
)


def kernel(x, emb, W1, b1, W2, b2):
    xT = x.T.astype(jnp.int32)
    counts1 = _hist0(xT)
    s1 = _mm(counts1, emb, emb, emb)
    counts2 = _hist1(xT)
    return _tail(counts2, emb, emb, emb, s1, W1.T, b1.reshape(1, _H),
                 W2.T, b2.reshape(1, _O))


# speedup vs baseline: 1.8500x; 1.1940x over previous
import functools

import jax
import jax.numpy as jnp
from jax import lax
from jax.experimental import pallas as pl
from jax.experimental.pallas import tpu as pltpu
from jax.experimental.pallas import tpu_sc as plsc

_VOCAB = 100000
_E = 64
_H = 256
_O = 5
_B = 16384
_L = 200
_NC = 2
_NS = 16
_NW = _NC * _NS
_VP = 102400
_HALF = _VP // 2
_COLS = -(-_L // _NW)
_KW = 6400
_KS = _HALF // _KW
_TAIL = _VOCAB - (_HALF + (_KS - 1) * _KW)


def _hist_body(xT, counts, idx_v, cnt_a, cnt_b, sem_a, sem_b):
    wid = lax.axis_index("s") * _NC + lax.axis_index("c")
    issued = [[], []]

    for j in range(_COLS):
        slot = j % 2
        buf, sem = (cnt_a, sem_a) if slot == 0 else (cnt_b, sem_b)
        l = wid + j * _NW
        cond = l < _L

        @pl.when(cond)
        def _(buf=buf, sem=sem, l=l):
            if issued[slot]:
                issued[slot][-1][0].wait()
                issued[slot][-1][2] = cond

            @plsc.parallel_loop(0, _HALF, 16, unroll=8)
            def _zero(i):
                buf[pl.ds(i, 16)] = jnp.zeros((16,), jnp.int32)

            pltpu.sync_copy(xT.at[l], idx_v)

            @plsc.parallel_loop(0, _B, 16, unroll=8)
            def _scat(i):
                v = idx_v[pl.ds(i, 16)]
                hi = (v >= _HALF).astype(jnp.int32)
                a = v - hi * _HALF
                w = jnp.int32(1) << (hi << 4)
                plsc.addupdate_scatter(buf, [a], w)

            issued[slot].append([pltpu.async_copy(buf, counts.at[l], sem),
                                 cond, None])

    for slot in (0, 1):
        for desc, icond, dcond in issued[slot]:
            rem = icond if dcond is None else icond & jnp.logical_not(dcond)

            @pl.when(rem)
            def _(desc=desc):
                desc.wait()


_hist = functools.partial(
    pl.kernel,
    out_type=jax.ShapeDtypeStruct((_L, _HALF), jnp.int32),
    mesh=plsc.VectorSubcoreMesh(core_axis_name="c", subcore_axis_name="s"),
    scratch_types=[
        pltpu.VMEM((_B,), jnp.int32),
        pltpu.VMEM((_HALF,), jnp.int32),
        pltpu.VMEM((_HALF,), jnp.int32),
        pltpu.SemaphoreType.DMA,
        pltpu.SemaphoreType.DMA,
    ],
    compiler_params=pltpu.CompilerParams(needs_layout_passes=False),
)(_hist_body)


def _tail_body(counts_ref, emb_lo_ref, emb_hi_ref, emb_tail_ref,
               w1t_ref, b1_ref, w2t_ref, b2_ref, out_ref, acc_ref):
    k = pl.program_id(0)

    @pl.when(k == 0)
    def _():
        acc_ref[...] = jnp.zeros_like(acc_ref)

    blk = counts_ref[...]
    lo = (blk & 0xFFFF).astype(jnp.float32)
    hi = (blk >> 16).astype(jnp.float32)
    acc_ref[...] += jnp.dot(lo, emb_lo_ref[...],
                            preferred_element_type=jnp.float32)

    @pl.when(k < _KS - 1)
    def _():
        acc_ref[...] += jnp.dot(hi, emb_hi_ref[...],
                                preferred_element_type=jnp.float32)

    @pl.when(k == _KS - 1)
    def _():
        acc_ref[...] += jnp.dot(hi[:, :_TAIL], emb_tail_ref[...],
                                preferred_element_type=jnp.float32)
        s = acc_ref[...] * (1.0 / _B)
        h1 = jnp.tanh(s @ w1t_ref[...] + b1_ref[...])
        h2 = h1 @ w2t_ref[...] + b2_ref[...]
        m = jnp.max(h2, axis=0, keepdims=True)
        lse = jnp.log(jnp.sum(jnp.exp(h2 - m), axis=0, keepdims=True)) + m
        out_ref[...] = h2 - lse


_tail = pl.pallas_call(
    _tail_body,
    grid=(_KS,),
    in_specs=[
        pl.BlockSpec((_L, _KW), lambda k: (0, k)),
        pl.BlockSpec((_KW, _E), lambda k: (k, 0)),
        pl.BlockSpec((_KW, _E), lambda k: (jnp.minimum(k + _KS, 2 * _KS - 2), 0)),
        pl.BlockSpec((_TAIL, _E), lambda k: (_VOCAB // _TAIL - 1, 0)),
        pl.BlockSpec((_E, _H), lambda k: (0, 0)),
        pl.BlockSpec((1, _H), lambda k: (0, 0)),
        pl.BlockSpec((_H, _O), lambda k: (0, 0)),
        pl.BlockSpec((1, _O), lambda k: (0, 0)),
    ],
    out_specs=pl.BlockSpec((_L, _O), lambda k: (0, 0)),
    out_shape=jax.ShapeDtypeStruct((_L, _O), jnp.float32),
    scratch_shapes=[pltpu.VMEM((_L, _E), jnp.float32)],
)


def kernel(x, emb, W1, b1, W2, b2):
    xT = x.T.astype(jnp.int32)
    counts = _hist(xT)
    return _tail(counts, emb, emb, emb, W1.T, b1.reshape(1, _H),
                 W2.T, b2.reshape(1, _O))
